# baseline (speedup 1.0000x reference)
import functools
import math

import jax
import jax.numpy as jnp
from jax import lax
from jax.experimental import pallas as pl
from jax.experimental.pallas import tpu as pltpu
from jax.experimental.pallas import tpu_sc as plsc

CUT = 0.5
EPS = 1e-08
NH = 4

_NC = 2
_NS = 16
_NW = _NC * _NS
_C = 128


def _sc_gather(h, row, col):
    N, D = h.shape
    E = row.shape[0]
    n_chunks = E // _C
    mesh = plsc.VectorSubcoreMesh(core_axis_name="c", subcore_axis_name="s")

    @functools.partial(
        pl.kernel, mesh=mesh,
        out_type=[jax.ShapeDtypeStruct((E, D), jnp.float32),
                  jax.ShapeDtypeStruct((E, D), jnp.float32)],
        scratch_types=[pltpu.VMEM((_C,), jnp.int32),
                       pltpu.VMEM((_C,), jnp.int32),
                       pltpu.VMEM((_C, D), jnp.float32),
                       pltpu.VMEM((_C, D), jnp.float32),
                       pltpu.SemaphoreType.DMA,
                       pltpu.SemaphoreType.DMA],
    )
    def k(h_hbm, row_hbm, col_hbm, src_out, tgt_out,
          idx_r, idx_c, rows_s, rows_t, sem_s, sem_t):
        wid = lax.axis_index("s") * _NC + lax.axis_index("c")

        def body(kk, carry):
            chunk = kk * _NW + wid

            @pl.when(chunk < n_chunks)
            def _():
                base = chunk * _C
                pltpu.sync_copy(row_hbm.at[pl.ds(base, _C)], idx_r)
                pltpu.sync_copy(col_hbm.at[pl.ds(base, _C)], idx_c)
                cp_s = pltpu.async_copy(h_hbm.at[idx_r], rows_s, sem_s)
                cp_t = pltpu.async_copy(h_hbm.at[idx_c], rows_t, sem_t)
                cp_s.wait()
                cp_t.wait()
                pltpu.sync_copy(rows_s, src_out.at[pl.ds(base, _C)])
                pltpu.sync_copy(rows_t, tgt_out.at[pl.ds(base, _C)])
            return carry

        lax.fori_loop(0, pl.cdiv(n_chunks, _NW), body, 0)

    return k(h, row, col)


def _sc_scatter(row, ef, zef):
    E, Hd = ef.shape
    NP = zef.shape[0]
    n_chunks = E // _C
    rows_per_sub = NP // _NS
    mesh = plsc.VectorSubcoreMesh(core_axis_name="c", subcore_axis_name="s")

    @functools.partial(
        pl.kernel, mesh=mesh,
        out_type=jax.ShapeDtypeStruct((2 * NP, Hd), jnp.float32),
        scratch_types=[pltpu.VMEM((_C,), jnp.int32),
                       pltpu.VMEM((_C, Hd), jnp.float32),
                       pltpu.VMEM_SHARED((NP, Hd), jnp.float32)],
    )
    def k(row_hbm, ef_hbm, zef_hbm, oef, idx_v, ef_v, acc_ef):
        cid = lax.axis_index("c")
        sid = lax.axis_index("s")
        wid = sid * _NC + cid
        rbase = sid * rows_per_sub
        pltpu.sync_copy(zef_hbm.at[pl.ds(rbase, rows_per_sub)],
                        acc_ef.at[pl.ds(rbase, rows_per_sub)])
        plsc.subcore_barrier()

        def body(kk, carry):
            chunk = kk * _NW + wid

            @pl.when(chunk < n_chunks)
            def _():
                base = chunk * _C
                pltpu.sync_copy(row_hbm.at[pl.ds(base, _C)], idx_v)
                pltpu.sync_copy(ef_hbm.at[pl.ds(base, _C)], ef_v)
                pltpu.sync_copy(ef_v, acc_ef.at[idx_v], add=True)
            return carry

        lax.fori_loop(0, pl.cdiv(n_chunks, _NW), body, 0)
        plsc.subcore_barrier()
        obase = cid * NP + rbase
        pltpu.sync_copy(acc_ef.at[pl.ds(rbase, rows_per_sub)],
                        oef.at[pl.ds(obase, rows_per_sub)])

    return k(row, ef, zef)


def _celu(x, alpha=2.0):
    return jnp.where(x > 0, x, alpha * (jnp.exp(x / alpha) - 1.0))


def _edge_kernel(src_ref, tgt_ref, aux_ref,
                 f_w1a_ref, f_w1b_ref, f_b1_ref, f_w2_ref, f_b2_ref,
                 e_w1a_ref, e_w1b_ref, radp_ref, e_w1w_ref, e_b1_ref,
                 e_w2_ref, e_b2_ref,
                 sa_w1p_ref, sa_b1p_ref, sa_w2t_ref, sa_b2b_ref,
                 sp_wr_ref, sp_br_ref,
                 rbf_wp_ref, rbf_b_ref,
                 ef_out_ref, av_out_ref):
    src = src_ref[...]
    tgt = tgt_ref[...]
    aux = aux_ref[...]

    rbf_e = jnp.dot(aux, rbf_wp_ref[...],
                    preferred_element_type=jnp.float32) + rbf_b_ref[...]
    t = (jnp.dot(src, f_w1a_ref[...], preferred_element_type=jnp.float32)
         + jnp.dot(tgt, f_w1b_ref[...], preferred_element_type=jnp.float32)
         + f_b1_ref[...])
    wf = _celu(t)
    wf = jnp.dot(wf, f_w2_ref[...],
                 preferred_element_type=jnp.float32) + f_b2_ref[...]
    g = rbf_e * wf
    pre = (jnp.dot(src, e_w1a_ref[...], preferred_element_type=jnp.float32)
           + jnp.dot(tgt, e_w1b_ref[...], preferred_element_type=jnp.float32)
           + jnp.dot(aux, radp_ref[...], preferred_element_type=jnp.float32)
           + jnp.dot(g, e_w1w_ref[...], preferred_element_type=jnp.float32)
           + e_b1_ref[...])
    ef = _celu(pre)
    ef = _celu(jnp.dot(ef, e_w2_ref[...],
                       preferred_element_type=jnp.float32) + e_b2_ref[...])

    sem_b = (jnp.dot(_celu(jnp.dot(ef, sa_w1p_ref[...],
                                   preferred_element_type=jnp.float32)
                           + sa_b1p_ref[...]),
                     sa_w2t_ref[...], preferred_element_type=jnp.float32)
             + sa_b2b_ref[...])
    be = src.shape[0]
    ef = ef * sem_b * aux[:, 1:2]

    rep = jnp.dot(ef, sp_wr_ref[...],
                  preferred_element_type=jnp.float32) + sp_br_ref[...]
    cdn = aux[:, 2:5]
    til = jnp.concatenate([cdn, cdn, cdn, cdn], axis=1)
    av12 = rep * til
    av = jnp.concatenate([av12, jnp.zeros((be, 116), jnp.float32)], axis=1)

    ef_out_ref[...] = ef
    av_out_ref[...] = av


def _node_kernel(h_ref, agg0_ref, agg1_ref, av0_ref, av1_ref,
                 mu_w1_ref, mu_b1_ref, mu_w2_ref, mu_b2_ref,
                 n_w1a_ref, n_w1b_ref, n_w1c_ref, n_b1_ref,
                 n_w2_ref, n_b2_ref,
                 out_ref):
    h = h_ref[...]
    agg = agg0_ref[0] + agg0_ref[1] + agg1_ref[0] + agg1_ref[1]
    av = av0_ref[0] + av0_ref[1] + av1_ref[0] + av1_ref[1]
    sq = av * av
    norms = jnp.concatenate(
        [jnp.sqrt(sq[:, 3 * i:3 * i + 1] + sq[:, 3 * i + 1:3 * i + 2]
                  + sq[:, 3 * i + 2:3 * i + 3]) for i in range(NH)], axis=1)
    spat = _celu(jnp.dot(norms, mu_w1_ref[...],
                         preferred_element_type=jnp.float32) + mu_b1_ref[...])
    spat = _celu(jnp.dot(spat, mu_w2_ref[...],
                         preferred_element_type=jnp.float32) + mu_b2_ref[...])
    out = _celu(jnp.dot(h, n_w1a_ref[...], preferred_element_type=jnp.float32)
                + jnp.dot(agg, n_w1b_ref[...], preferred_element_type=jnp.float32)
                + jnp.dot(spat, n_w1c_ref[...], preferred_element_type=jnp.float32)
                + n_b1_ref[...])
    out_ref[...] = _celu(jnp.dot(out, n_w2_ref[...],
                                 preferred_element_type=jnp.float32)
                         + n_b2_ref[...])


def _full_spec():
    return pl.BlockSpec(lambda i: tuple(), None)


def kernel(h, edge_index, radial, coord_diff, rbf, e_w1, e_b1, e_w2, e_b2,
           n_w1, n_b1, n_w2, n_b2, sp_w, sp_b, sa_w1, sa_b1, sa_w2, sa_b2,
           rbf_w, rbf_b, f_w1, f_b1, f_w2, f_b2, mu_w1, mu_b1, mu_w2, mu_b2):
    N, D = h.shape
    E = edge_index.shape[1]
    H = e_w2.shape[0]
    KS = rbf.shape[1]
    row = edge_index[0]
    col = edge_index[1]

    f_w1a, f_w1b = f_w1[:D], f_w1[D:]
    e_w1a, e_w1b = e_w1[:D], e_w1[D:2 * D]
    e_w1r, e_w1w = e_w1[2 * D:2 * D + 1], e_w1[2 * D + 1:]
    n_w1a, n_w1b, n_w1c = n_w1[:D], n_w1[D:D + H], n_w1[D + H:]

    sa_w1p = jnp.zeros((H, H), jnp.float32).at[:, :NH].set(sa_w1)
    sa_b1p = jnp.zeros((H,), jnp.float32).at[:NH].set(sa_b1)
    sa_w2t = jnp.zeros((H, H), jnp.float32).at[:NH, :].set(
        jnp.broadcast_to(sa_w2, (NH, H)))
    sa_b2b = jnp.broadcast_to(sa_b2, (H,))
    rmat = jnp.zeros((NH, 12), jnp.float32)
    for i in range(NH):
        rmat = rmat.at[i, 3 * i:3 * i + 3].set(1.0)
    sp_wr = sp_w @ rmat
    sp_br = sp_b @ rmat

    euclid = (0.5 * (jnp.cos(jnp.sqrt(radial[:, 0])
                             * (math.pi / 2.0 * CUT)) + 1.0))[:, None]
    nrm = jnp.sqrt(jnp.sum(coord_diff * coord_diff, axis=1, keepdims=True))
    cdn = coord_diff / nrm + EPS
    aux = jnp.concatenate(
        [radial, euclid, cdn, jnp.zeros((E, 3), jnp.float32),
         rbf, jnp.zeros((E, 128 - 8 - KS), jnp.float32)], axis=1)
    radp = jnp.zeros((128, 128), jnp.float32).at[0, :].set(e_w1r[0])
    rbf_wp = jnp.zeros((128, 128), jnp.float32).at[8:8 + KS, :].set(rbf_w)

    BE = 1280
    NHALF = 2
    E2 = E // NHALF
    assert E2 % BE == 0 and E2 % _C == 0
    grid_e = E2 // BE

    def espec(width):
        return pl.BlockSpec((BE, width), lambda i: (i, 0))

    def wspec(arr):
        r = arr.ndim
        return pl.BlockSpec(arr.shape, lambda i, r=r: (0,) * r)

    e_weights = (f_w1a, f_w1b, f_b1, f_w2, f_b2,
                 e_w1a, e_w1b, radp, e_w1w, e_b1, e_w2, e_b2,
                 sa_w1p, sa_b1p, sa_w2t, sa_b2b, sp_wr, sp_br, rbf_wp, rbf_b)

    NP = ((N + 8 * _NS - 1) // (8 * _NS)) * (8 * _NS)
    zef = jnp.zeros((NP, H), jnp.float32)

    aggs, av_aggs = [], []
    for hh in range(NHALF):
        off = hh * grid_e

        def hspec(width, off=off):
            return pl.BlockSpec((BE, width), lambda i, off=off: (i + off, 0))

        edge_call = pl.pallas_call(
            _edge_kernel,
            grid=(grid_e,),
            in_specs=[espec(D), espec(D), hspec(128)]
                     + [wspec(a) for a in e_weights],
            out_specs=[espec(H), espec(128)],
            out_shape=[jax.ShapeDtypeStruct((E2, H), jnp.float32),
                       jax.ShapeDtypeStruct((E2, 128), jnp.float32)],
        )
        row_h = lax.dynamic_slice(row, (hh * E2,), (E2,))
        col_h = lax.dynamic_slice(col, (hh * E2,), (E2,))
        src, tgt = _sc_gather(h, row_h, col_h)
        ef, av = edge_call(src, tgt, aux, *e_weights)
        aggs.append(_sc_scatter(row_h, ef, zef).reshape(2, NP, H))
        av_aggs.append(_sc_scatter(row_h, av, zef).reshape(2, NP, 128))

    BN = 2000
    assert N % BN == 0
    grid_n = N // BN

    n_weights = (mu_w1, mu_b1, mu_w2, mu_b2,
                 n_w1a, n_w1b, n_w1c, n_b1, n_w2, n_b2)
    pspec = pl.BlockSpec((2, BN, 128), lambda i: (0, i, 0))
    out = pl.pallas_call(
        _node_kernel,
        grid=(grid_n,),
        in_specs=[pl.BlockSpec((BN, D), lambda i: (i, 0)),
                  pspec, pspec, pspec, pspec]
                 + [wspec(a) for a in n_weights],
        out_specs=pl.BlockSpec((BN, D), lambda i: (i, 0)),
        out_shape=jax.ShapeDtypeStruct((N, D), jnp.float32),
    )(h, aggs[0], aggs[1], av_aggs[0], av_aggs[1], *n_weights)
    return out


# speedup vs baseline: 1.3240x; 1.3240x over previous
import functools
import math

import jax
import jax.numpy as jnp
from jax import lax
from jax.experimental import pallas as pl
from jax.experimental.pallas import tpu as pltpu
from jax.experimental.pallas import tpu_sc as plsc

CUT = 0.5
EPS = 1e-08
NH = 4

_NC = 2
_NS = 16
_NW = _NC * _NS
_C = 128


def _sc_gather(h, row, col):
    N, D = h.shape
    E = row.shape[0]
    n_chunks = E // _C
    mesh = plsc.VectorSubcoreMesh(core_axis_name="c", subcore_axis_name="s")

    @functools.partial(
        pl.kernel, mesh=mesh,
        out_type=[jax.ShapeDtypeStruct((E, D), jnp.float32),
                  jax.ShapeDtypeStruct((E, D), jnp.float32)],
        scratch_types=[pltpu.VMEM((_C,), jnp.int32),
                       pltpu.VMEM((_C,), jnp.int32),
                       pltpu.VMEM((_C, D), jnp.float32),
                       pltpu.VMEM((_C, D), jnp.float32),
                       pltpu.SemaphoreType.DMA,
                       pltpu.SemaphoreType.DMA],
    )
    def k(h_hbm, row_hbm, col_hbm, src_out, tgt_out,
          idx_r, idx_c, rows_s, rows_t, sem_s, sem_t):
        wid = lax.axis_index("s") * _NC + lax.axis_index("c")

        def body(kk, carry):
            chunk = kk * _NW + wid

            @pl.when(chunk < n_chunks)
            def _():
                base = chunk * _C
                pltpu.sync_copy(row_hbm.at[pl.ds(base, _C)], idx_r)
                pltpu.sync_copy(col_hbm.at[pl.ds(base, _C)], idx_c)
                cp_s = pltpu.async_copy(h_hbm.at[idx_r], rows_s, sem_s)
                cp_t = pltpu.async_copy(h_hbm.at[idx_c], rows_t, sem_t)
                cp_s.wait()
                cp_t.wait()
                pltpu.sync_copy(rows_s, src_out.at[pl.ds(base, _C)])
                pltpu.sync_copy(rows_t, tgt_out.at[pl.ds(base, _C)])
            return carry

        lax.fori_loop(0, pl.cdiv(n_chunks, _NW), body, 0)

    return k(h, row, col)


def _sc_scatter(row, ef, zef):
    E, Hd = ef.shape
    NP = zef.shape[0]
    n_chunks = E // _C
    rows_per_sub = NP // _NS
    mesh = plsc.VectorSubcoreMesh(core_axis_name="c", subcore_axis_name="s")

    @functools.partial(
        pl.kernel, mesh=mesh,
        out_type=jax.ShapeDtypeStruct((2 * NP, Hd), jnp.float32),
        scratch_types=[pltpu.VMEM((_C,), jnp.int32),
                       pltpu.VMEM((_C, Hd), jnp.float32),
                       pltpu.VMEM_SHARED((NP, Hd), jnp.float32)],
    )
    def k(row_hbm, ef_hbm, zef_hbm, oef, idx_v, ef_v, acc_ef):
        cid = lax.axis_index("c")
        sid = lax.axis_index("s")
        wid = sid * _NC + cid
        rbase = sid * rows_per_sub
        pltpu.sync_copy(zef_hbm.at[pl.ds(rbase, rows_per_sub)],
                        acc_ef.at[pl.ds(rbase, rows_per_sub)])
        plsc.subcore_barrier()

        def body(kk, carry):
            chunk = kk * _NW + wid

            @pl.when(chunk < n_chunks)
            def _():
                base = chunk * _C
                pltpu.sync_copy(row_hbm.at[pl.ds(base, _C)], idx_v)
                pltpu.sync_copy(ef_hbm.at[pl.ds(base, _C)], ef_v)
                pltpu.sync_copy(ef_v, acc_ef.at[idx_v], add=True)
            return carry

        lax.fori_loop(0, pl.cdiv(n_chunks, _NW), body, 0)
        plsc.subcore_barrier()
        obase = cid * NP + rbase
        pltpu.sync_copy(acc_ef.at[pl.ds(rbase, rows_per_sub)],
                        oef.at[pl.ds(obase, rows_per_sub)])

    return k(row, ef, zef)


def _celu(x, alpha=2.0):
    return jnp.where(x > 0, x, alpha * (jnp.exp(x / alpha) - 1.0))


def _edge_kernel(src_ref, tgt_ref, radial_ref, cdn_ref, rbf_ref,
                 f_w1a_ref, f_w1b_ref, f_b1_ref, f_w2_ref, f_b2_ref,
                 e_w1a_ref, e_w1b_ref, e_w1r_ref, e_w1w_ref, e_b1_ref,
                 e_w2_ref, e_b2_ref,
                 sa_w1p_ref, sa_b1p_ref, sa_w2t_ref, sa_b2b_ref,
                 sp_wr_ref, sp_br_ref,
                 rbf_w_ref, rbf_b_ref,
                 ef_out_ref, av_out_ref):
    src = src_ref[...]
    tgt = tgt_ref[...]
    radial = radial_ref[...]
    rbf = rbf_ref[...]

    rbf_e = jnp.dot(rbf, rbf_w_ref[...],
                    preferred_element_type=jnp.float32) + rbf_b_ref[...]
    t = (jnp.dot(src, f_w1a_ref[...], preferred_element_type=jnp.float32)
         + jnp.dot(tgt, f_w1b_ref[...], preferred_element_type=jnp.float32)
         + f_b1_ref[...])
    wf = _celu(t)
    wf = jnp.dot(wf, f_w2_ref[...],
                 preferred_element_type=jnp.float32) + f_b2_ref[...]
    g = rbf_e * wf
    pre = (jnp.dot(src, e_w1a_ref[...], preferred_element_type=jnp.float32)
           + jnp.dot(tgt, e_w1b_ref[...], preferred_element_type=jnp.float32)
           + radial * e_w1r_ref[...]
           + jnp.dot(g, e_w1w_ref[...], preferred_element_type=jnp.float32)
           + e_b1_ref[...])
    ef = _celu(pre)
    ef = _celu(jnp.dot(ef, e_w2_ref[...],
                       preferred_element_type=jnp.float32) + e_b2_ref[...])

    sem_b = (jnp.dot(_celu(jnp.dot(ef, sa_w1p_ref[...],
                                   preferred_element_type=jnp.float32)
                           + sa_b1p_ref[...]),
                     sa_w2t_ref[...], preferred_element_type=jnp.float32)
             + sa_b2b_ref[...])
    be = src.shape[0]
    rad_b = jnp.broadcast_to(radial, (be, sem_b.shape[1]))
    euclid = 0.5 * (jnp.cos(jnp.sqrt(rad_b) * (math.pi / 2.0 * CUT)) + 1.0)
    ef = ef * sem_b * euclid

    rep = jnp.dot(ef, sp_wr_ref[...],
                  preferred_element_type=jnp.float32) + sp_br_ref[...]
    cdn = cdn_ref[...]
    til = jnp.concatenate([cdn, cdn, cdn, cdn], axis=1)
    av12 = rep * til
    av = jnp.concatenate([av12, jnp.zeros((be, 116), jnp.float32)], axis=1)

    ef_out_ref[...] = ef
    av_out_ref[...] = av


def _node_kernel(h_ref, agg0_ref, agg1_ref, av0_ref, av1_ref,
                 mu_w1_ref, mu_b1_ref, mu_w2_ref, mu_b2_ref,
                 n_w1a_ref, n_w1b_ref, n_w1c_ref, n_b1_ref,
                 n_w2_ref, n_b2_ref,
                 out_ref):
    h = h_ref[...]
    agg = agg0_ref[0] + agg0_ref[1] + agg1_ref[0] + agg1_ref[1]
    av = av0_ref[0] + av0_ref[1] + av1_ref[0] + av1_ref[1]
    sq = av * av
    norms = jnp.concatenate(
        [jnp.sqrt(sq[:, 3 * i:3 * i + 1] + sq[:, 3 * i + 1:3 * i + 2]
                  + sq[:, 3 * i + 2:3 * i + 3]) for i in range(NH)], axis=1)
    spat = _celu(jnp.dot(norms, mu_w1_ref[...],
                         preferred_element_type=jnp.float32) + mu_b1_ref[...])
    spat = _celu(jnp.dot(spat, mu_w2_ref[...],
                         preferred_element_type=jnp.float32) + mu_b2_ref[...])
    out = _celu(jnp.dot(h, n_w1a_ref[...], preferred_element_type=jnp.float32)
                + jnp.dot(agg, n_w1b_ref[...], preferred_element_type=jnp.float32)
                + jnp.dot(spat, n_w1c_ref[...], preferred_element_type=jnp.float32)
                + n_b1_ref[...])
    out_ref[...] = _celu(jnp.dot(out, n_w2_ref[...],
                                 preferred_element_type=jnp.float32)
                         + n_b2_ref[...])


def _full_spec():
    return pl.BlockSpec(lambda i: tuple(), None)


def kernel(h, edge_index, radial, coord_diff, rbf, e_w1, e_b1, e_w2, e_b2,
           n_w1, n_b1, n_w2, n_b2, sp_w, sp_b, sa_w1, sa_b1, sa_w2, sa_b2,
           rbf_w, rbf_b, f_w1, f_b1, f_w2, f_b2, mu_w1, mu_b1, mu_w2, mu_b2):
    N, D = h.shape
    E = edge_index.shape[1]
    H = e_w2.shape[0]
    KS = rbf.shape[1]
    row = edge_index[0]
    col = edge_index[1]

    f_w1a, f_w1b = f_w1[:D], f_w1[D:]
    e_w1a, e_w1b = e_w1[:D], e_w1[D:2 * D]
    e_w1r, e_w1w = e_w1[2 * D:2 * D + 1], e_w1[2 * D + 1:]
    n_w1a, n_w1b, n_w1c = n_w1[:D], n_w1[D:D + H], n_w1[D + H:]

    sa_w1p = jnp.zeros((H, H), jnp.float32).at[:, :NH].set(sa_w1)
    sa_b1p = jnp.zeros((H,), jnp.float32).at[:NH].set(sa_b1)
    sa_w2t = jnp.zeros((H, H), jnp.float32).at[:NH, :].set(
        jnp.broadcast_to(sa_w2, (NH, H)))
    sa_b2b = jnp.broadcast_to(sa_b2, (H,))
    rmat = jnp.zeros((NH, 12), jnp.float32)
    for i in range(NH):
        rmat = rmat.at[i, 3 * i:3 * i + 3].set(1.0)
    sp_wr = sp_w @ rmat
    sp_br = sp_b @ rmat

    nrm = jnp.sqrt(jnp.sum(coord_diff * coord_diff, axis=1, keepdims=True))
    cdn = coord_diff / nrm + EPS

    BE = 1280
    NHALF = 2
    E2 = E // NHALF
    assert E2 % BE == 0 and E2 % _C == 0
    grid_e = E2 // BE

    def espec(width):
        return pl.BlockSpec((BE, width), lambda i: (i, 0))

    def wspec(arr):
        r = arr.ndim
        return pl.BlockSpec(arr.shape, lambda i, r=r: (0,) * r)

    e_weights = (f_w1a, f_w1b, f_b1, f_w2, f_b2,
                 e_w1a, e_w1b, e_w1r, e_w1w, e_b1, e_w2, e_b2,
                 sa_w1p, sa_b1p, sa_w2t, sa_b2b, sp_wr, sp_br, rbf_w, rbf_b)

    NP = ((N + 8 * _NS - 1) // (8 * _NS)) * (8 * _NS)
    zef = jnp.zeros((NP, H), jnp.float32)

    aggs, av_aggs = [], []
    for hh in range(NHALF):
        off = hh * grid_e

        def hspec(width, off=off):
            return pl.BlockSpec((BE, width), lambda i, off=off: (i + off, 0))

        edge_call = pl.pallas_call(
            _edge_kernel,
            grid=(grid_e,),
            in_specs=[espec(D), espec(D), hspec(1), hspec(3), hspec(KS)]
                     + [wspec(a) for a in e_weights],
            out_specs=[espec(H), espec(128)],
            out_shape=[jax.ShapeDtypeStruct((E2, H), jnp.float32),
                       jax.ShapeDtypeStruct((E2, 128), jnp.float32)],
        )
        row_h = lax.dynamic_slice(row, (hh * E2,), (E2,))
        col_h = lax.dynamic_slice(col, (hh * E2,), (E2,))
        src, tgt = _sc_gather(h, row_h, col_h)
        ef, av = edge_call(src, tgt, radial, cdn, rbf, *e_weights)
        aggs.append(_sc_scatter(row_h, ef, zef).reshape(2, NP, H))
        av_aggs.append(_sc_scatter(row_h, av, zef).reshape(2, NP, 128))

    BN = 2000
    assert N % BN == 0
    grid_n = N // BN

    n_weights = (mu_w1, mu_b1, mu_w2, mu_b2,
                 n_w1a, n_w1b, n_w1c, n_b1, n_w2, n_b2)
    pspec = pl.BlockSpec((2, BN, 128), lambda i: (0, i, 0))
    out = pl.pallas_call(
        _node_kernel,
        grid=(grid_n,),
        in_specs=[pl.BlockSpec((BN, D), lambda i: (i, 0)),
                  pspec, pspec, pspec, pspec]
                 + [wspec(a) for a in n_weights],
        out_specs=pl.BlockSpec((BN, D), lambda i: (i, 0)),
        out_shape=jax.ShapeDtypeStruct((N, D), jnp.float32),
    )(h, aggs[0], aggs[1], av_aggs[0], av_aggs[1], *n_weights)
    return out
